# baseline (speedup 1.0000x reference)
import functools

import jax
import jax.numpy as jnp
from jax import lax
from jax.experimental import pallas as pl
from jax.experimental.pallas import tpu as pltpu
from jax.experimental.pallas import tpu_sc as plsc

B = 16384
D = 64
L = 16
_NC = 2
_NS = 16
_NW = _NC * _NS
_NROW = 1000000
_RPW = 31232
_CHR = 512
_NCH = _RPW // _CHR
_EXTRA0 = 999424
_TAIL0 = 999936
_K = 16384
_IC = 2048
_NIV = _IC // L


def _sc_stream_extract(table_t, tail, i, j, neg_j):
    mesh = plsc.VectorSubcoreMesh(core_axis_name="c", subcore_axis_name="s")

    @functools.partial(
        pl.kernel,
        mesh=mesh,
        compiler_params=pltpu.CompilerParams(needs_layout_passes=False),
        out_type=[jax.ShapeDtypeStruct((3 * B * D,), jnp.float32)],
        scratch_types=[
            pltpu.VMEM((_IC,), jnp.int32),
            pltpu.VMEM((_K + L,), jnp.int32),
            pltpu.VMEM((_K + L,), jnp.int32),
            pltpu.VMEM((_K + L,), jnp.int32),
            pltpu.VMEM((_K + L,), jnp.int32),
            pltpu.VMEM((D, _CHR + 1), jnp.float32),
            pltpu.VMEM((D * D,), jnp.float32),
            pltpu.VMEM((L * D,), jnp.float32),
            pltpu.SemaphoreType.DMA,
            pltpu.SemaphoreType.DMA,
        ],
    )
    def k(tab_hbm, tail_hbm, i_hbm, j_hbm, n_hbm, out_hbm,
          idxb, wlr, wls, clr, cls, buf, tail_v, stage, sem_s, sem_o):
        wid = lax.axis_index("s") * _NC + lax.axis_index("c")
        w0 = wid == 0
        row_lo = wid * _RPW
        row_hi = row_lo + _RPW
        lane = lax.broadcasted_iota(jnp.int32, (L,), 0)

        @pl.when(w0)
        def _load_tail():
            pltpu.sync_copy(tail_hbm, tail_v)

        def extract_entries(cc, c_lo, from_tail):
            def ext(t2, carry):
                rl = clr[pl.ds(t2 * L, L)]
                sl = cls[pl.ds(t2 * L, L)]
                for q in range(L):
                    @pl.when(t2 * L + q < cc)
                    def _fire(q=q, rl=rl, sl=sl):
                        rloc = rl[q]
                        for f in range(D // L):
                            if from_tail:
                                vals = tail_v[pl.ds(rloc * D + f * L, L)]
                            else:
                                cols = lane * 0 + rloc
                                vals = plsc.load_gather(
                                    buf, [lane + f * L, cols])
                            stage[pl.ds(q * D + f * L, L)] = vals
                        pltpu.async_copy(
                            stage.at[pl.ds(q * D, D)],
                            out_hbm.at[pl.ds(sl[q] * D, D)], sem_o)
                for q in range(L):
                    @pl.when(t2 * L + q < cc)
                    def _drain(q=q, sl=sl):
                        pltpu.make_async_copy(
                            stage.at[pl.ds(q * D, D)],
                            out_hbm.at[pl.ds(sl[q] * D, D)], sem_o).wait()
                return carry

            lax.fori_loop(0, (cc + L - 1) // L, ext, 0)

        def compact_and_extract(cnt, c_lo, c_hi, from_tail):
            def compact(t, cc):
                rows = wlr[pl.ds(t * L, L)]
                slots = wls[pl.ds(t * L, L)]
                valid = (t * L + lane) < cnt
                m = valid & (rows >= c_lo) & (rows < c_hi)
                plsc.store_compressed(clr.at[pl.ds(cc, L)], rows - c_lo, mask=m)
                plsc.store_compressed(cls.at[pl.ds(cc, L)], slots, mask=m)
                return cc + plsc.all_reduce_population_count(m)[0]

            cc = lax.fori_loop(0, (cnt + L - 1) // L, compact, 0)
            extract_entries(cc, c_lo, from_tail)

        rva = [jnp.int32(0), jnp.int32(0), jnp.int32(0)]
        for rnd in range(4):
            cnt = jnp.int32(0)
            new_rva = []
            for a, src in enumerate((i_hbm, j_hbm, n_hbm)):
                def chunk_scan(ci, carry, src=src, a=a):
                    cnt, rv = carry

                    @pl.when(rv == jnp.int32(8))
                    def _ld(ci=ci, src=src):
                        pltpu.sync_copy(src.at[pl.ds(ci * _IC, _IC)], idxb)

                    def vec(t, c2, a=a, ci=ci):
                        cnt, stop = c2
                        v = idxb[pl.ds(t * L, L)]
                        m = (v >= row_lo) & (v < row_hi)
                        if True:
                            m = m | ((v >= _EXTRA0) & w0)
                        full = cnt + L > _K
                        can = jnp.logical_not(stop) & jnp.logical_not(full)
                        mm = m & can
                        plsc.store_compressed(wlr.at[pl.ds(cnt, L)], v, mask=mm)
                        slotv = a * B + ci * _IC + t * L + lane
                        plsc.store_compressed(wls.at[pl.ds(cnt, L)], slotv, mask=mm)
                        pc = plsc.all_reduce_population_count(mm)[0]
                        return (cnt + pc, stop | full)

                    trips = jnp.where(rv == jnp.int32(8), _NIV, 0)
                    pre = cnt
                    cnt2, stop2 = lax.fori_loop(0, trips, vec,
                                                (cnt, jnp.bool_(False)))
                    cnt3 = jnp.where(stop2, pre, cnt2)
                    rv2 = jnp.where(stop2 & (rv == jnp.int32(8)),
                                    ci, rv)
                    return (cnt3, rv2)

                rv0 = rva[a]

                def outer(ci, carry, rv0=rv0, src=src, a=a):
                    cnt, rv = carry
                    do = ci >= rv0
                    c2 = lax.cond(
                        do,
                        lambda c: chunk_scan(ci, c),
                        lambda c: c,
                        (cnt, rv))
                    return c2

                cnt, rvn = lax.fori_loop(
                    0, 8, outer, (cnt, jnp.int32(8)))
                new_rva.append(jnp.where(rvn == jnp.int32(8),
                                         jnp.int32(8), rvn))
            rva = new_rva

            def chunk_body(kk, carry, cnt=cnt):
                r0 = pl.multiple_of(
                    jnp.where(kk < _NCH, row_lo + kk * _CHR,
                              jnp.int32(_EXTRA0)), 128)
                cp = pltpu.async_copy(
                    tab_hbm.at[:, pl.ds(r0, _CHR)],
                    buf.at[:, pl.ds(0, _CHR)], sem_s)
                cp.wait()
                compact_and_extract(cnt, r0, r0 + _CHR, False)
                return carry

            trips = jnp.where(cnt > 0,
                              jnp.where(w0, _NCH + 1, _NCH), 0)
            lax.fori_loop(0, trips, chunk_body, 0)

            @pl.when(w0 & (cnt > 0))
            def _tail_chunk(cnt=cnt):
                compact_and_extract(cnt, jnp.int32(_TAIL0),
                                    jnp.int32(_NROW), True)

    return k(table_t, tail, i, j, neg_j)


_TC_BLK = 512
_PR = 3 * B * D // 128


def _tc_loss(packed):
    def body(ui_ref, uj_ref, un_ref, out_ref):
        @pl.when(pl.program_id(0) == 0)
        def _init():
            out_ref[0, 0] = 0.0

        r_idx = lax.broadcasted_iota(jnp.int32, (128, 128), 0)
        c_idx = lax.broadcasted_iota(jnp.int32, (128, 128), 1)
        sel = (r_idx // D == c_idx).astype(jnp.float32)
        a = ui_ref[...]
        s_pos = jnp.dot(a * uj_ref[...], sel,
                        preferred_element_type=jnp.float32)
        s_neg = jnp.dot(a * un_ref[...], sel,
                        preferred_element_type=jnp.float32)

        def softplus(x):
            return jnp.maximum(x, 0.0) + jnp.log1p(jnp.exp(-jnp.abs(x)))

        valid = (lax.broadcasted_iota(jnp.int32, (_TC_BLK, 128), 1)
                 < 2).astype(jnp.float32)
        contrib = valid * (softplus(-s_pos) + softplus(s_neg))
        out_ref[0, 0] += jnp.sum(contrib) * (1.0 / B)

    nblk = B * D // 128 // _TC_BLK
    out = pl.pallas_call(
        body,
        grid=(nblk,),
        in_specs=[
            pl.BlockSpec((_TC_BLK, 128), lambda g: (g, 0)),
            pl.BlockSpec((_TC_BLK, 128), lambda g, n=nblk: (g + n, 0)),
            pl.BlockSpec((_TC_BLK, 128), lambda g, n=nblk: (g + 2 * n, 0)),
        ],
        out_specs=pl.BlockSpec((1, 1), lambda g: (0, 0),
                               memory_space=pltpu.SMEM),
        out_shape=jax.ShapeDtypeStruct((1, 1), jnp.float32),
    )(packed, packed, packed)
    return out[0, 0]


def kernel(table, i, j, neg_j):
    tail = table[_TAIL0:, :].reshape(D * D)
    rows = _sc_stream_extract(
        table.T, tail,
        i.astype(jnp.int32), j.astype(jnp.int32), neg_j.astype(jnp.int32),
    )[0]
    return _tc_loss(rows.reshape(_PR, 128))


# speedup vs baseline: 1.6353x; 1.6353x over previous
import functools

import jax
import jax.numpy as jnp
from jax import lax
from jax.experimental import pallas as pl
from jax.experimental.pallas import tpu as pltpu
from jax.experimental.pallas import tpu_sc as plsc

B = 16384
D = 64
L = 16
_NC = 2
_NS = 16
_NW = _NC * _NS
_NROW = 1000000
_RPW = 31232
_CHR = 512
_NCH = _RPW // _CHR
_EXTRA0 = 999424
_TAIL0 = 999936
_K = 11264
_IC = 2048
_NIV = _IC // L


def _sc_stream_extract(table_t, tail, i, j, neg_j):
    mesh = plsc.VectorSubcoreMesh(core_axis_name="c", subcore_axis_name="s")

    @functools.partial(
        pl.kernel,
        mesh=mesh,
        compiler_params=pltpu.CompilerParams(needs_layout_passes=False),
        out_type=[jax.ShapeDtypeStruct((3 * B * D,), jnp.float32)],
        scratch_types=[
            pltpu.VMEM((_IC,), jnp.int32),
            pltpu.VMEM((_K + L,), jnp.int32),
            pltpu.VMEM((_K + L,), jnp.int32),
            pltpu.VMEM((_K + L,), jnp.int32),
            pltpu.VMEM((_K + L,), jnp.int32),
            pltpu.VMEM((2, D, _CHR), jnp.float32),
            pltpu.VMEM((D * D,), jnp.float32),
            pltpu.VMEM((L * 65,), jnp.float32),
            pltpu.VMEM((L * D,), jnp.float32),
            pltpu.SemaphoreType.DMA,
            pltpu.SemaphoreType.DMA,
        ],
    )
    def k(tab_hbm, tail_hbm, i_hbm, j_hbm, n_hbm, out_hbm,
          idxb, wlr, wls, clr, cls, buf, tail_v, stage, stage2, sem_s, sem_o):
        wid = lax.axis_index("s") * _NC + lax.axis_index("c")
        w0 = wid == 0
        row_lo = wid * _RPW
        row_hi = row_lo + _RPW
        lane = lax.broadcasted_iota(jnp.int32, (L,), 0)

        @pl.when(w0)
        def _load_tail():
            pltpu.sync_copy(tail_hbm, tail_v)

        def extract_entries(cc, par, from_tail):
            skew = lane * 65

            def ext(t2, carry):
                rl = clr[pl.ds(t2 * L, L)]
                sl = cls[pl.ds(t2 * L, L)]
                valid = (t2 * L + lane) < cc
                pvec = lane * 0 + par
                for f in range(D):
                    if from_tail:
                        vals = plsc.load_gather(
                            tail_v, [rl * D + f], mask=valid)
                    else:
                        vals = plsc.load_gather(
                            buf, [pvec, lane * 0 + f, rl], mask=valid)
                    plsc.store_scatter(stage, [skew + f], vals, mask=valid)
                for q in range(L):
                    for f4 in range(D // L):
                        stage2[pl.ds(q * D + f4 * L, L)] = (
                            stage[pl.ds(q * 65 + f4 * L, L)])
                for q in range(L):
                    @pl.when(t2 * L + q < cc)
                    def _fire(q=q, sl=sl):
                        pltpu.async_copy(
                            stage2.at[pl.ds(q * D, D)],
                            out_hbm.at[pl.ds(sl[q] * D, D)], sem_o)
                for q in range(L):
                    @pl.when(t2 * L + q < cc)
                    def _drain(q=q, sl=sl):
                        pltpu.make_async_copy(
                            stage2.at[pl.ds(q * D, D)],
                            out_hbm.at[pl.ds(sl[q] * D, D)], sem_o).wait()
                return carry

            lax.fori_loop(0, (cc + L - 1) // L, ext, 0)

        def compact_only(cnt, c_lo, c_hi):
            def compact(t, cc):
                rows = wlr[pl.ds(t * L, L)]
                slots = wls[pl.ds(t * L, L)]
                valid = (t * L + lane) < cnt
                m = valid & (rows >= c_lo) & (rows < c_hi)
                plsc.store_compressed(clr.at[pl.ds(cc, L)], rows - c_lo, mask=m)
                plsc.store_compressed(cls.at[pl.ds(cc, L)], slots, mask=m)
                return cc + plsc.all_reduce_population_count(m)[0]

            return lax.fori_loop(0, (cnt + L - 1) // L, compact, 0)

        def chunk_r0(kk):
            return pl.multiple_of(
                jnp.where(kk < _NCH, row_lo + kk * _CHR,
                          jnp.int32(_EXTRA0)), 128)

        def enqueue_chunk(kk):
            r0 = chunk_r0(kk)
            pltpu.async_copy(
                tab_hbm.at[:, pl.ds(r0, _CHR)],
                buf.at[kk % 2, :, pl.ds(0, _CHR)], sem_s)

        def wait_chunk(kk):
            r0 = chunk_r0(kk)
            pltpu.make_async_copy(
                tab_hbm.at[:, pl.ds(r0, _CHR)],
                buf.at[kk % 2, :, pl.ds(0, _CHR)], sem_s).wait()

        rva = [jnp.int32(0), jnp.int32(0), jnp.int32(0)]
        for rnd in range(6):
            cnt = jnp.int32(0)
            new_rva = []
            for a, src in enumerate((i_hbm, j_hbm, n_hbm)):
                def chunk_scan(ci, carry, src=src, a=a):
                    cnt, rv = carry

                    @pl.when(rv == jnp.int32(8))
                    def _ld(ci=ci, src=src):
                        pltpu.sync_copy(src.at[pl.ds(ci * _IC, _IC)], idxb)

                    def vec(t, c2, a=a, ci=ci):
                        cnt, stop = c2
                        v = idxb[pl.ds(t * L, L)]
                        m = (v >= row_lo) & (v < row_hi)
                        if True:
                            m = m | ((v >= _EXTRA0) & w0)
                        full = cnt + L > _K
                        can = jnp.logical_not(stop) & jnp.logical_not(full)
                        mm = m & can
                        plsc.store_compressed(wlr.at[pl.ds(cnt, L)], v, mask=mm)
                        slotv = a * B + ci * _IC + t * L + lane
                        plsc.store_compressed(wls.at[pl.ds(cnt, L)], slotv, mask=mm)
                        pc = plsc.all_reduce_population_count(mm)[0]
                        return (cnt + pc, stop | full)

                    trips = jnp.where(rv == jnp.int32(8), _NIV, 0)
                    pre = cnt
                    cnt2, stop2 = lax.fori_loop(0, trips, vec,
                                                (cnt, jnp.bool_(False)))
                    cnt3 = jnp.where(stop2, pre, cnt2)
                    rv2 = jnp.where(stop2 & (rv == jnp.int32(8)),
                                    ci, rv)
                    return (cnt3, rv2)

                rv0 = rva[a]

                def outer(ci, carry, rv0=rv0, src=src, a=a):
                    cnt, rv = carry
                    do = ci >= rv0
                    c2 = lax.cond(
                        do,
                        lambda c: chunk_scan(ci, c),
                        lambda c: c,
                        (cnt, rv))
                    return c2

                cnt, rvn = lax.fori_loop(
                    0, 8, outer, (cnt, jnp.int32(8)))
                new_rva.append(jnp.where(rvn == jnp.int32(8),
                                         jnp.int32(8), rvn))
            rva = new_rva

            trips = jnp.where(cnt > 0,
                              jnp.where(w0, _NCH + 1, _NCH), 0)

            @pl.when(trips > 0)
            def _prime():
                enqueue_chunk(jnp.int32(0))

            def chunk_body(kk, carry, cnt=cnt, trips=trips):
                @pl.when(kk + 1 < trips)
                def _prefetch(kk=kk):
                    enqueue_chunk(kk + 1)

                r0 = chunk_r0(kk)
                cc = compact_only(cnt, r0, r0 + _CHR)
                wait_chunk(kk)
                extract_entries(cc, kk % 2, False)
                return carry

            lax.fori_loop(0, trips, chunk_body, 0)

            @pl.when(w0 & (cnt > 0))
            def _tail_chunk(cnt=cnt):
                cc = compact_only(cnt, jnp.int32(_TAIL0), jnp.int32(_NROW))
                extract_entries(cc, jnp.int32(0), True)

    return k(table_t, tail, i, j, neg_j)


_TC_BLK = 512
_PR = 3 * B * D // 128


def _tc_loss(packed):
    def body(ui_ref, uj_ref, un_ref, out_ref):
        @pl.when(pl.program_id(0) == 0)
        def _init():
            out_ref[0, 0] = 0.0

        r_idx = lax.broadcasted_iota(jnp.int32, (128, 128), 0)
        c_idx = lax.broadcasted_iota(jnp.int32, (128, 128), 1)
        sel = (r_idx // D == c_idx).astype(jnp.float32)
        a = ui_ref[...]
        s_pos = jnp.dot(a * uj_ref[...], sel,
                        preferred_element_type=jnp.float32)
        s_neg = jnp.dot(a * un_ref[...], sel,
                        preferred_element_type=jnp.float32)

        def softplus(x):
            return jnp.maximum(x, 0.0) + jnp.log1p(jnp.exp(-jnp.abs(x)))

        valid = (lax.broadcasted_iota(jnp.int32, (_TC_BLK, 128), 1)
                 < 2).astype(jnp.float32)
        contrib = valid * (softplus(-s_pos) + softplus(s_neg))
        out_ref[0, 0] += jnp.sum(contrib) * (1.0 / B)

    nblk = B * D // 128 // _TC_BLK
    out = pl.pallas_call(
        body,
        grid=(nblk,),
        in_specs=[
            pl.BlockSpec((_TC_BLK, 128), lambda g: (g, 0)),
            pl.BlockSpec((_TC_BLK, 128), lambda g, n=nblk: (g + n, 0)),
            pl.BlockSpec((_TC_BLK, 128), lambda g, n=nblk: (g + 2 * n, 0)),
        ],
        out_specs=pl.BlockSpec((1, 1), lambda g: (0, 0),
                               memory_space=pltpu.SMEM),
        out_shape=jax.ShapeDtypeStruct((1, 1), jnp.float32),
    )(packed, packed, packed)
    return out[0, 0]


def kernel(table, i, j, neg_j):
    tail = table[_TAIL0:, :].reshape(D * D)
    rows = _sc_stream_extract(
        table.T, tail,
        i.astype(jnp.int32), j.astype(jnp.int32), neg_j.astype(jnp.int32),
    )[0]
    return _tc_loss(rows.reshape(_PR, 128))
